# baseline (speedup 1.0000x reference)
import numpy as np
import jax
import jax.numpy as jnp
from jax.experimental import pallas as pl

L = 12
DIM = 2
T = 2 ** 19
BASE = 16
DESIRED = 2048
PRIMES = (1, 2654435761, 805459861)

_B = np.exp((np.log(DESIRED) - np.log(BASE)) / (L - 1))
_RES = tuple(int(np.floor(BASE * (_B ** l))) for l in range(L))
_CORNERS = tuple((dx, dy, dz) for dx in (0, 1) for dy in (0, 1) for dz in (0, 1))

_S = 2048
_AX = 16


def _core_body(frac_ref, dts_ref, r0d_ref, r1d_ref, r0c_ref, r1c_ref,
               w1d0_ref, w1d1_ref, w1c0_ref, w1c1_ref, w2d_ref, w2c_ref,
               out_ref):
    S = _S
    frac = frac_ref[...]
    row = jax.lax.broadcasted_iota(jnp.int32, (8 * L, 3 * _AX), 0)
    col = jax.lax.broadcasted_iota(jnp.int32, (8 * L, 3 * _AX), 1)
    f96 = []
    for a in range(3):
        Ea = ((col // _AX == a) & (col % _AX == row % L)).astype(jnp.float32)
        f96.append(jnp.dot(Ea, frac, preferred_element_type=jnp.float32))
    r1 = jax.lax.broadcasted_iota(jnp.int32, (8 * L, 1), 0)
    c_id = r1 // L
    wx = jnp.where((c_id // 4) % 2 == 1, f96[0], 1.0 - f96[0])
    wy = jnp.where((c_id // 2) % 2 == 1, f96[1], 1.0 - f96[1])
    wz = jnp.where(c_id % 2 == 1, f96[2], 1.0 - f96[2])
    w96 = wx * wy * wz

    rr = jax.lax.broadcasted_iota(jnp.int32, (L, 8 * L), 0)
    cc = jax.lax.broadcasted_iota(jnp.int32, (L, 8 * L), 1)
    R = (cc % L == rr).astype(jnp.float32)

    f0d = jnp.dot(R, w96 * r0d_ref[...], preferred_element_type=jnp.float32)
    f1d = jnp.dot(R, w96 * r1d_ref[...], preferred_element_type=jnp.float32)
    f0c = jnp.dot(R, w96 * r0c_ref[...], preferred_element_type=jnp.float32)
    f1c = jnp.dot(R, w96 * r1c_ref[...], preferred_element_type=jnp.float32)

    hd = jnp.maximum(
        jnp.dot(w1d0_ref[...], f0d, preferred_element_type=jnp.float32)
        + jnp.dot(w1d1_ref[...], f1d, preferred_element_type=jnp.float32), 0.0)
    sig = jnp.exp(jnp.clip(
        jnp.dot(w2d_ref[...], hd, preferred_element_type=jnp.float32),
        -15.0, 15.0))
    hc = jnp.maximum(
        jnp.dot(w1c0_ref[...], f0c, preferred_element_type=jnp.float32)
        + jnp.dot(w1c1_ref[...], f1c, preferred_element_type=jnp.float32), 0.0)
    logits = jnp.dot(w2c_ref[...], hc, preferred_element_type=jnp.float32)
    rgb = 1.0 / (1.0 + jnp.exp(-logits))

    sdt = sig * dts_ref[...].reshape(1, S)
    alpha = 1.0 - jnp.exp(-sdt)
    out_ref[...] = jnp.concatenate(
        [sdt, alpha, rgb, jnp.zeros((3, S), jnp.float32)], axis=0)


def kernel(rays_o, rays_d, t_starts, t_ends, ray_indices, table_d, table_c,
           w1_d, w2_d, w1_c, w2_c):
    n_rays = rays_o.shape[0]
    n = t_starts.shape[0]
    t_mid = (t_starts + t_ends) * 0.5
    dts = t_ends - t_starts

    xs_a = []
    for a in range(3):
        x = rays_o[:, a][ray_indices] + rays_d[:, a][ray_indices] * t_mid
        xs_a.append((jnp.clip(x, -1.0, 1.0) + 1.0) / 2.0)

    frac_rows = [jnp.zeros((n,), jnp.float32)] * (3 * _AX)
    p0 = {}
    for a in range(3):
        for l in range(L):
            pos = xs_a[a] * np.float32(_RES[l])
            p0f = jnp.floor(pos)
            frac_rows[a * _AX + l] = pos - p0f
            p0[(a, l)] = p0f.astype(jnp.uint32)
    frac48 = jnp.stack(frac_rows)

    idx_rows = []
    for dx, dy, dz in _CORNERS:
        for l in range(L):
            h = ((p0[(0, l)] + np.uint32(dx)) * np.uint32(PRIMES[0])) \
                ^ ((p0[(1, l)] + np.uint32(dy)) * np.uint32(PRIMES[1])) \
                ^ ((p0[(2, l)] + np.uint32(dz)) * np.uint32(PRIMES[2]))
            idx_rows.append((h & np.uint32(T - 1)).astype(jnp.int32)
                            + np.int32(l * T))
    idx96 = jnp.stack(idx_rows)

    r0d = table_d[:, :, 0].reshape(L * T)[idx96]
    r1d = table_d[:, :, 1].reshape(L * T)[idx96]
    r0c = table_c[:, :, 0].reshape(L * T)[idx96]
    r1c = table_c[:, :, 1].reshape(L * T)[idx96]

    w1d0 = w1_d[0::2, :].T
    w1d1 = w1_d[1::2, :].T
    w1c0 = w1_c[0::2, :].T
    w1c1 = w1_c[1::2, :].T
    w2dT = w2_d.T
    w2cT = w2_c.T

    grid = (n // _S,)
    big = lambda r: pl.BlockSpec((r, _S), lambda i: (0, i))
    out8 = pl.pallas_call(
        _core_body,
        grid=grid,
        in_specs=[
            big(3 * _AX),
            pl.BlockSpec((_S,), lambda i: (i,)),
            big(8 * L), big(8 * L), big(8 * L), big(8 * L),
            pl.BlockSpec((32, L), lambda i: (0, 0)),
            pl.BlockSpec((32, L), lambda i: (0, 0)),
            pl.BlockSpec((32, L), lambda i: (0, 0)),
            pl.BlockSpec((32, L), lambda i: (0, 0)),
            pl.BlockSpec((1, 32), lambda i: (0, 0)),
            pl.BlockSpec((3, 32), lambda i: (0, 0)),
        ],
        out_specs=pl.BlockSpec((8, _S), lambda i: (0, i)),
        out_shape=jax.ShapeDtypeStruct((8, n), jnp.float32),
    )(frac48, dts, r0d, r1d, r0c, r1c,
      w1d0, w1d1, w1c0, w1c1, w2dT, w2cT)

    sdt = out8[0]
    alphas = out8[1]

    cum = jnp.concatenate([jnp.zeros((1,), jnp.float32), jnp.cumsum(sdt)])
    first_idx = jnp.searchsorted(ray_indices,
                                 jnp.arange(n_rays, dtype=ray_indices.dtype))
    offsets = cum[first_idx]
    within_excl = cum[:-1] - offsets[ray_indices]
    trans = jnp.exp(-within_excl)
    weights = trans * alphas
    acc = jax.ops.segment_sum(weights, ray_indices, num_segments=n_rays)
    cols = [jax.ops.segment_sum(weights * out8[2 + ch], ray_indices,
                                num_segments=n_rays) for ch in range(3)]
    color = jnp.stack(cols, axis=1)
    color = jnp.clip(color + (1.0 - acc[:, None]), 0.0, 1.0)
    alpha = jnp.clip(acc, 0.0, 1.0)
    return (color, alpha)


# speedup vs baseline: 23.1379x; 23.1379x over previous
import numpy as np
import jax
import jax.numpy as jnp
from jax.experimental import pallas as pl

L = 12
DIM = 2
T = 2 ** 19
BASE = 16
DESIRED = 2048
PRIMES = (1, 2654435761, 805459861)

_B = np.exp((np.log(DESIRED) - np.log(BASE)) / (L - 1))
_RES = tuple(int(np.floor(BASE * (_B ** l))) for l in range(L))
_CORNERS = tuple((dx, dy, dz) for dx in (0, 1) for dy in (0, 1) for dz in (0, 1))

_S = 2048
_AX = 16


def _core_body(frac_ref, ts_ref, te_ref, rows_d_ref, rows_c_ref,
               w1d_ref, w2d_ref, w1c_ref, w2c_ref,
               sdt_ref, alpha_ref, rgb_ref):
    frac = frac_ref[...]
    row = jax.lax.broadcasted_iota(jnp.int32, (3 * _AX, 2 * L), 0)
    col = jax.lax.broadcasted_iota(jnp.int32, (3 * _AX, 2 * L), 1)
    f24 = []
    for a in range(3):
        Ea = ((row // _AX == a) & (row % _AX == col // 2)
              & (row % _AX < L)).astype(jnp.float32)
        f24.append(jnp.dot(frac, Ea, preferred_element_type=jnp.float32))

    feat_d = jnp.zeros((frac.shape[0], 2 * L), jnp.float32)
    feat_c = jnp.zeros((frac.shape[0], 2 * L), jnp.float32)
    for c, (dx, dy, dz) in enumerate(_CORNERS):
        wx = f24[0] if dx else 1.0 - f24[0]
        wy = f24[1] if dy else 1.0 - f24[1]
        wz = f24[2] if dz else 1.0 - f24[2]
        w = wx * wy * wz
        feat_d = feat_d + w * rows_d_ref[:, c * 2 * L:(c + 1) * 2 * L]
        feat_c = feat_c + w * rows_c_ref[:, c * 2 * L:(c + 1) * 2 * L]

    hd = jnp.maximum(jnp.dot(feat_d, w1d_ref[...],
                             preferred_element_type=jnp.float32), 0.0)
    sig = jnp.exp(jnp.clip(jnp.dot(hd, w2d_ref[...],
                                   preferred_element_type=jnp.float32),
                           -15.0, 15.0))[:, 0]
    hc = jnp.maximum(jnp.dot(feat_c, w1c_ref[...],
                             preferred_element_type=jnp.float32), 0.0)
    logits = jnp.dot(hc, w2c_ref[...], preferred_element_type=jnp.float32)
    rgb_ref[...] = 1.0 / (1.0 + jnp.exp(-logits))

    sdt = sig * (te_ref[...] - ts_ref[...])
    sdt_ref[...] = sdt
    alpha_ref[...] = 1.0 - jnp.exp(-sdt)


def kernel(rays_o, rays_d, t_starts, t_ends, ray_indices, table_d, table_c,
           w1_d, w2_d, w1_c, w2_c):
    n_rays = rays_o.shape[0]
    n = t_starts.shape[0]
    t_mid = (t_starts + t_ends)[:, None] * 0.5
    xs = rays_o[ray_indices] + rays_d[ray_indices] * t_mid
    xs_n = (jnp.clip(xs, -1.0, 1.0) + 1.0) / 2.0

    res_f = jnp.asarray(_RES, jnp.float32)[None, :, None]
    pos = xs_n[:, None, :] * res_f
    p0f = jnp.floor(pos)
    p0 = p0f.astype(jnp.uint32)
    fr = pos - p0f
    pad = jnp.zeros((n, _AX - L), jnp.float32)
    frac48 = jnp.concatenate(
        [jnp.concatenate([fr[:, :, a], pad], axis=1) for a in range(3)],
        axis=1)

    lofs = (jnp.arange(L, dtype=jnp.int32) * T)[None, :]
    td = table_d.reshape(L * T, DIM)
    tc = table_c.reshape(L * T, DIM)
    rows_d, rows_c = [], []
    for dx, dy, dz in _CORNERS:
        cx = p0[:, :, 0] + np.uint32(dx)
        cy = p0[:, :, 1] + np.uint32(dy)
        cz = p0[:, :, 2] + np.uint32(dz)
        h = (cx * np.uint32(PRIMES[0])) ^ (cy * np.uint32(PRIMES[1])) \
            ^ (cz * np.uint32(PRIMES[2]))
        idx = (h & np.uint32(T - 1)).astype(jnp.int32) + lofs
        rows_d.append(td[idx].reshape(n, 2 * L))
        rows_c.append(tc[idx].reshape(n, 2 * L))
    rows_d = jnp.concatenate(rows_d, axis=1)
    rows_c = jnp.concatenate(rows_c, axis=1)

    grid = (n // _S,)
    sdt, alphas, rgbs = pl.pallas_call(
        _core_body,
        grid=grid,
        in_specs=[
            pl.BlockSpec((_S, 3 * _AX), lambda i: (i, 0)),
            pl.BlockSpec((_S,), lambda i: (i,)),
            pl.BlockSpec((_S,), lambda i: (i,)),
            pl.BlockSpec((_S, 16 * L), lambda i: (i, 0)),
            pl.BlockSpec((_S, 16 * L), lambda i: (i, 0)),
            pl.BlockSpec((2 * L, 32), lambda i: (0, 0)),
            pl.BlockSpec((32, 1), lambda i: (0, 0)),
            pl.BlockSpec((2 * L, 32), lambda i: (0, 0)),
            pl.BlockSpec((32, 3), lambda i: (0, 0)),
        ],
        out_specs=[
            pl.BlockSpec((_S,), lambda i: (i,)),
            pl.BlockSpec((_S,), lambda i: (i,)),
            pl.BlockSpec((_S, 3), lambda i: (i, 0)),
        ],
        out_shape=[
            jax.ShapeDtypeStruct((n,), jnp.float32),
            jax.ShapeDtypeStruct((n,), jnp.float32),
            jax.ShapeDtypeStruct((n, 3), jnp.float32),
        ],
    )(frac48, t_starts, t_ends, rows_d, rows_c, w1_d, w2_d, w1_c, w2_c)

    cum = jnp.concatenate([jnp.zeros((1,), jnp.float32), jnp.cumsum(sdt)])
    first_idx = jnp.searchsorted(ray_indices,
                                 jnp.arange(n_rays, dtype=ray_indices.dtype))
    offsets = cum[first_idx]
    within_excl = cum[:-1] - offsets[ray_indices]
    trans = jnp.exp(-within_excl)
    weights = trans * alphas
    color = jax.ops.segment_sum(weights[:, None] * rgbs, ray_indices,
                                num_segments=n_rays)
    acc = jax.ops.segment_sum(weights, ray_indices, num_segments=n_rays)
    color = jnp.clip(color + (1.0 - acc[:, None]), 0.0, 1.0)
    alpha = jnp.clip(acc, 0.0, 1.0)
    return (color, alpha)


# speedup vs baseline: 43.7904x; 1.8926x over previous
import numpy as np
import jax
import jax.numpy as jnp
from jax.experimental import pallas as pl

L = 12
DIM = 2
T = 2 ** 19
BASE = 16
DESIRED = 2048
PRIMES = (1, 2654435761, 805459861)

_B = np.exp((np.log(DESIRED) - np.log(BASE)) / (L - 1))
_RES = tuple(int(np.floor(BASE * (_B ** l))) for l in range(L))
_CORNERS = tuple((dx, dy, dz) for dx in (0, 1) for dy in (0, 1) for dz in (0, 1))

_S = 2048
_AX = 16


def _core_body(frac_ref, ts_ref, te_ref, rows_d_ref, rows_c_ref,
               w1d_ref, w2d_ref, w1c_ref, w2c_ref,
               sdt_ref, alpha_ref, rgb_ref):
    frac = frac_ref[...]
    row = jax.lax.broadcasted_iota(jnp.int32, (3 * _AX, 2 * L), 0)
    col = jax.lax.broadcasted_iota(jnp.int32, (3 * _AX, 2 * L), 1)
    f24 = []
    for a in range(3):
        Ea = ((row // _AX == a) & (row % _AX == col // 2)
              & (row % _AX < L)).astype(jnp.float32)
        f24.append(jnp.dot(frac, Ea, preferred_element_type=jnp.float32))

    feat_d = jnp.zeros((frac.shape[0], 2 * L), jnp.float32)
    feat_c = jnp.zeros((frac.shape[0], 2 * L), jnp.float32)
    for c, (dx, dy, dz) in enumerate(_CORNERS):
        wx = f24[0] if dx else 1.0 - f24[0]
        wy = f24[1] if dy else 1.0 - f24[1]
        wz = f24[2] if dz else 1.0 - f24[2]
        w = wx * wy * wz
        feat_d = feat_d + w * rows_d_ref[:, c * 2 * L:(c + 1) * 2 * L]
        feat_c = feat_c + w * rows_c_ref[:, c * 2 * L:(c + 1) * 2 * L]

    hd = jnp.maximum(jnp.dot(feat_d, w1d_ref[...],
                             preferred_element_type=jnp.float32), 0.0)
    sig = jnp.exp(jnp.clip(jnp.dot(hd, w2d_ref[...],
                                   preferred_element_type=jnp.float32),
                           -15.0, 15.0))[:, 0]
    hc = jnp.maximum(jnp.dot(feat_c, w1c_ref[...],
                             preferred_element_type=jnp.float32), 0.0)
    logits = jnp.dot(hc, w2c_ref[...], preferred_element_type=jnp.float32)
    rgb_ref[...] = 1.0 / (1.0 + jnp.exp(-logits))

    sdt = sig * (te_ref[...] - ts_ref[...])
    sdt_ref[...] = sdt
    alpha_ref[...] = 1.0 - jnp.exp(-sdt)


def kernel(rays_o, rays_d, t_starts, t_ends, ray_indices, table_d, table_c,
           w1_d, w2_d, w1_c, w2_c):
    n_rays = rays_o.shape[0]
    n = t_starts.shape[0]
    t_mid = (t_starts + t_ends)[:, None] * 0.5
    xs = rays_o[ray_indices] + rays_d[ray_indices] * t_mid
    xs_n = (jnp.clip(xs, -1.0, 1.0) + 1.0) / 2.0

    res_f = jnp.asarray(_RES, jnp.float32)[None, :, None]
    pos = xs_n[:, None, :] * res_f
    p0f = jnp.floor(pos)
    p0 = p0f.astype(jnp.uint32)
    fr = pos - p0f
    pad = jnp.zeros((n, _AX - L), jnp.float32)
    frac48 = jnp.concatenate(
        [jnp.concatenate([fr[:, :, a], pad], axis=1) for a in range(3)],
        axis=1)

    rows_d, rows_c = [], []
    for dx, dy, dz in _CORNERS:
        for l in range(L):
            h = ((p0[:, l, 0] + np.uint32(dx)) * np.uint32(PRIMES[0])) \
                ^ ((p0[:, l, 1] + np.uint32(dy)) * np.uint32(PRIMES[1])) \
                ^ ((p0[:, l, 2] + np.uint32(dz)) * np.uint32(PRIMES[2]))
            idx = (h & np.uint32(T - 1)).astype(jnp.int32)
            rows_d.append(table_d[l][idx])
            rows_c.append(table_c[l][idx])
    rows_d = jnp.concatenate(rows_d, axis=1)
    rows_c = jnp.concatenate(rows_c, axis=1)

    grid = (n // _S,)
    sdt, alphas, rgbs = pl.pallas_call(
        _core_body,
        grid=grid,
        in_specs=[
            pl.BlockSpec((_S, 3 * _AX), lambda i: (i, 0)),
            pl.BlockSpec((_S,), lambda i: (i,)),
            pl.BlockSpec((_S,), lambda i: (i,)),
            pl.BlockSpec((_S, 16 * L), lambda i: (i, 0)),
            pl.BlockSpec((_S, 16 * L), lambda i: (i, 0)),
            pl.BlockSpec((2 * L, 32), lambda i: (0, 0)),
            pl.BlockSpec((32, 1), lambda i: (0, 0)),
            pl.BlockSpec((2 * L, 32), lambda i: (0, 0)),
            pl.BlockSpec((32, 3), lambda i: (0, 0)),
        ],
        out_specs=[
            pl.BlockSpec((_S,), lambda i: (i,)),
            pl.BlockSpec((_S,), lambda i: (i,)),
            pl.BlockSpec((_S, 3), lambda i: (i, 0)),
        ],
        out_shape=[
            jax.ShapeDtypeStruct((n,), jnp.float32),
            jax.ShapeDtypeStruct((n,), jnp.float32),
            jax.ShapeDtypeStruct((n, 3), jnp.float32),
        ],
    )(frac48, t_starts, t_ends, rows_d, rows_c, w1_d, w2_d, w1_c, w2_c)

    cum = jnp.concatenate([jnp.zeros((1,), jnp.float32), jnp.cumsum(sdt)])
    first_idx = jnp.searchsorted(ray_indices,
                                 jnp.arange(n_rays, dtype=ray_indices.dtype))
    offsets = cum[first_idx]
    within_excl = cum[:-1] - offsets[ray_indices]
    trans = jnp.exp(-within_excl)
    weights = trans * alphas
    color = jax.ops.segment_sum(weights[:, None] * rgbs, ray_indices,
                                num_segments=n_rays)
    acc = jax.ops.segment_sum(weights, ray_indices, num_segments=n_rays)
    color = jnp.clip(color + (1.0 - acc[:, None]), 0.0, 1.0)
    alpha = jnp.clip(acc, 0.0, 1.0)
    return (color, alpha)
